# baseline (speedup 1.0000x reference)
import functools

import jax
import jax.numpy as jnp
from jax import lax
from jax.experimental import pallas as pl
from jax.experimental.pallas import tpu as pltpu
from jax.experimental.pallas import tpu_sc as plsc

_TOPK = 8

_SORT8 = [(0, 1), (2, 3), (4, 5), (6, 7),
          (0, 2), (1, 3), (4, 6), (5, 7),
          (1, 2), (5, 6),
          (0, 4), (1, 5), (2, 6), (3, 7),
          (1, 4), (3, 6),
          (2, 4), (3, 5),
          (3, 4)]
_BMERGE8 = [(0, 4), (1, 5), (2, 6), (3, 7),
            (0, 2), (1, 3), (4, 6), (5, 7),
            (0, 1), (2, 3), (4, 5), (6, 7)]


def _sort8(v):
    v = list(v)
    for a, b in _SORT8:
        v[a], v[b] = jnp.maximum(v[a], v[b]), jnp.minimum(v[a], v[b])
    return v


def _merge_top8(A, B):
    c = [jnp.maximum(A[i], B[7 - i]) for i in range(8)]
    for a, b in _BMERGE8:
        c[a], c[b] = jnp.maximum(c[a], c[b]), jnp.minimum(c[a], c[b])
    return c


def _matmul_pack_body(x_ref, w_ref, b_ref, key_ref):
    lt = jax.lax.dot_general(
        w_ref[...], x_ref[...], (((1,), (1,)), ((), ())),
        preferred_element_type=jnp.float32,
    )
    lt = lt + b_ref[...]
    s = jax.lax.bitcast_convert_type(lt, jnp.int32)
    u = jax.lax.bitcast_convert_type(
        s ^ ((s >> 31) | jnp.int32(-(2 ** 31))), jnp.uint32
    )
    eid = jax.lax.broadcasted_iota(jnp.uint32, lt.shape, 0)
    key_ref[...] = (u & jnp.uint32(0xFFFFFFC0)) | (jnp.uint32(63) - eid)


def _packed_keys_t(x, W, b, tile):
    n, d = x.shape
    e = W.shape[0]
    return pl.pallas_call(
        _matmul_pack_body,
        grid=(n // tile,),
        in_specs=[
            pl.BlockSpec((tile, d), lambda i: (i, 0)),
            pl.BlockSpec((e, d), lambda i: (0, 0)),
            pl.BlockSpec((e, 1), lambda i: (0, 0)),
        ],
        out_specs=pl.BlockSpec((e, tile), lambda i: (0, i)),
        out_shape=jax.ShapeDtypeStruct((e, n), jnp.uint32),
    )(x, W, b.reshape(e, 1))


def _make_sc_topk(n, e):
    info = plsc.get_sparse_core_info()
    nc, ns, nl = info.num_cores, info.num_subcores, info.num_lanes
    nw = nc * ns
    assert n % (nw * nl) == 0 and e == 64
    tok_w = n // nw
    ngroups = tok_w // nl
    mesh = plsc.VectorSubcoreMesh(core_axis_name="c", subcore_axis_name="s")

    @functools.partial(
        pl.kernel, mesh=mesh,
        out_type=[
            jax.ShapeDtypeStruct((_TOPK, n), jnp.float32),
            jax.ShapeDtypeStruct((_TOPK, n), jnp.int32),
        ],
        scratch_types=[
            pltpu.VMEM((e, tok_w), jnp.uint32),
            pltpu.VMEM((_TOPK, tok_w), jnp.float32),
            pltpu.VMEM((_TOPK, tok_w), jnp.int32),
        ],
    )
    def sc_topk(key_hbm, gt_hbm, it_hbm, key_v, g_v, i_v):
        wid = lax.axis_index("s") * nc + lax.axis_index("c")
        base = wid * tok_w
        pltpu.sync_copy(key_hbm.at[:, pl.ds(base, tok_w)], key_v)

        def group_body(g, carry):
            off = g * nl

            top = _sort8([key_v[t, pl.ds(off, nl)] for t in range(8)])
            for j in range(1, 8):
                top = _merge_top8(
                    top, _sort8([key_v[8 * j + t, pl.ds(off, nl)]
                                 for t in range(8)])
                )

            vals, idxs = [], []
            for k in top:
                idxs.append((jnp.uint32(63) - (k & jnp.uint32(63)))
                            .astype(jnp.int32))
                vu = (k & jnp.uint32(0xFFFFFFC0)) | jnp.uint32(32)
                pos = vu >= jnp.uint32(0x80000000)
                sb = jnp.where(pos, vu ^ jnp.uint32(0x80000000), ~vu)
                vals.append(jax.lax.bitcast_convert_type(sb, jnp.float32))

            m = vals[0]
            exps = [jnp.exp(v - m) for v in vals]
            denom = exps[0]
            for s in exps[1:]:
                denom = denom + s
            inv = 1.0 / denom
            for k in range(_TOPK):
                g_v[k, pl.ds(off, nl)] = exps[k] * inv
                i_v[k, pl.ds(off, nl)] = idxs[k]
            return carry

        lax.fori_loop(0, ngroups, group_body, 0)
        pltpu.sync_copy(g_v, gt_hbm.at[:, pl.ds(base, tok_w)])
        pltpu.sync_copy(i_v, it_hbm.at[:, pl.ds(base, tok_w)])

    return sc_topk


def kernel(x, W, b):
    n, d = x.shape
    e = W.shape[0]
    tile = 1024 if n % 1024 == 0 else n
    nchunks = 2 if n % (2 * 8192) == 0 else 1
    cs = n // nchunks
    sc = _make_sc_topk(cs, e)
    outs = []
    for c in range(nchunks):
        keys = _packed_keys_t(x[c * cs:(c + 1) * cs], W, b, tile)
        outs.append(sc(keys))
    gt = jnp.concatenate([g for g, _ in outs], axis=1)
    it = jnp.concatenate([i for _, i in outs], axis=1)
    return gt.T, it.T.astype(jnp.int64)


# speedup vs baseline: 2.6683x; 2.6683x over previous
import functools

import jax
import jax.numpy as jnp
from jax import lax
from jax.experimental import pallas as pl
from jax.experimental.pallas import tpu as pltpu
from jax.experimental.pallas import tpu_sc as plsc

_TOPK = 8

_SORT8 = [(0, 1), (2, 3), (4, 5), (6, 7),
          (0, 2), (1, 3), (4, 6), (5, 7),
          (1, 2), (5, 6),
          (0, 4), (1, 5), (2, 6), (3, 7),
          (1, 4), (3, 6),
          (2, 4), (3, 5),
          (3, 4)]
_BMERGE8 = [(0, 4), (1, 5), (2, 6), (3, 7),
            (0, 2), (1, 3), (4, 6), (5, 7),
            (0, 1), (2, 3), (4, 5), (6, 7)]


def _sort8(v):
    v = list(v)
    for a, b in _SORT8:
        v[a], v[b] = jnp.maximum(v[a], v[b]), jnp.minimum(v[a], v[b])
    return v


def _merge_top8(A, B):
    c = [jnp.maximum(A[i], B[7 - i]) for i in range(8)]
    for a, b in _BMERGE8:
        c[a], c[b] = jnp.maximum(c[a], c[b]), jnp.minimum(c[a], c[b])
    return c


def _matmul_pack_body(x_ref, w_ref, b_ref, key_ref):
    lt = jax.lax.dot_general(
        w_ref[...], x_ref[...], (((1,), (1,)), ((), ())),
        preferred_element_type=jnp.float32,
    )
    lt = lt + b_ref[...]
    s = jax.lax.bitcast_convert_type(lt, jnp.int32)
    u = jax.lax.bitcast_convert_type(
        s ^ ((s >> 31) | jnp.int32(-(2 ** 31))), jnp.uint32
    )
    eid = jax.lax.broadcasted_iota(jnp.uint32, lt.shape, 0)
    key_ref[...] = (u & jnp.uint32(0xFFFFFFC0)) | (jnp.uint32(63) - eid)


def _packed_keys_t(x, W, b, tile, row0, rows):
    n, d = x.shape
    e = W.shape[0]
    off = row0 // tile
    return pl.pallas_call(
        _matmul_pack_body,
        grid=(rows // tile,),
        in_specs=[
            pl.BlockSpec((tile, d), lambda i: (off + i, 0)),
            pl.BlockSpec((e, d), lambda i: (0, 0)),
            pl.BlockSpec((e, 1), lambda i: (0, 0)),
        ],
        out_specs=pl.BlockSpec((e, tile), lambda i: (0, i)),
        out_shape=jax.ShapeDtypeStruct((e, rows), jnp.uint32),
    )(x, W, b.reshape(e, 1))


def _make_sc_topk(n, e):
    info = plsc.get_sparse_core_info()
    nc, ns, nl = info.num_cores, info.num_subcores, info.num_lanes
    nw = nc * ns
    assert n % (nw * nl) == 0 and e == 64
    tok_w = n // nw
    ngroups = tok_w // nl
    mesh = plsc.VectorSubcoreMesh(core_axis_name="c", subcore_axis_name="s")

    @functools.partial(
        pl.kernel, mesh=mesh,
        out_type=[
            jax.ShapeDtypeStruct((_TOPK, n), jnp.float32),
            jax.ShapeDtypeStruct((_TOPK, n), jnp.int32),
        ],
        scratch_types=[
            pltpu.VMEM((e, tok_w), jnp.uint32),
            pltpu.VMEM((_TOPK, tok_w), jnp.float32),
            pltpu.VMEM((_TOPK, tok_w), jnp.int32),
        ],
    )
    def sc_topk(key_hbm, gt_hbm, it_hbm, key_v, g_v, i_v):
        wid = lax.axis_index("s") * nc + lax.axis_index("c")
        base = wid * tok_w
        pltpu.sync_copy(key_hbm.at[:, pl.ds(base, tok_w)], key_v)

        def group_body(g, carry):
            off = g * nl

            top = _sort8([key_v[t, pl.ds(off, nl)] for t in range(8)])
            for j in range(1, 8):
                top = _merge_top8(
                    top, _sort8([key_v[8 * j + t, pl.ds(off, nl)]
                                 for t in range(8)])
                )

            vals, idxs = [], []
            for k in top:
                idxs.append((jnp.uint32(63) - (k & jnp.uint32(63)))
                            .astype(jnp.int32))
                vu = (k & jnp.uint32(0xFFFFFFC0)) | jnp.uint32(32)
                pos = vu >= jnp.uint32(0x80000000)
                sb = jnp.where(pos, vu ^ jnp.uint32(0x80000000), ~vu)
                vals.append(jax.lax.bitcast_convert_type(sb, jnp.float32))

            m = vals[0]
            exps = [jnp.exp(v - m) for v in vals]
            denom = exps[0]
            for s in exps[1:]:
                denom = denom + s
            inv = 1.0 / denom
            for k in range(_TOPK):
                g_v[k, pl.ds(off, nl)] = exps[k] * inv
                i_v[k, pl.ds(off, nl)] = idxs[k]
            return carry

        lax.fori_loop(0, ngroups, group_body, 0)
        pltpu.sync_copy(g_v, gt_hbm.at[:, pl.ds(base, tok_w)])
        pltpu.sync_copy(i_v, it_hbm.at[:, pl.ds(base, tok_w)])

    return sc_topk


def kernel(x, W, b):
    n, d = x.shape
    e = W.shape[0]
    tile = 1024 if n % 1024 == 0 else n
    nchunks = 2 if n % (2 * 8192) == 0 else 1
    cs = n // nchunks
    sc = _make_sc_topk(cs, e)
    outs = []
    for c in range(nchunks):
        keys = _packed_keys_t(x, W, b, tile, c * cs, cs)
        outs.append(sc(keys))
    gt = jnp.concatenate([g for g, _ in outs], axis=1)
    it = jnp.concatenate([i for _, i in outs], axis=1)
    return gt.T, it.T.astype(jnp.int64)


# speedup vs baseline: 2.7047x; 1.0137x over previous
import functools

import jax
import jax.numpy as jnp
from jax import lax
from jax.experimental import pallas as pl
from jax.experimental.pallas import tpu as pltpu
from jax.experimental.pallas import tpu_sc as plsc

_TOPK = 8

_SORT8 = [(0, 1), (2, 3), (4, 5), (6, 7),
          (0, 2), (1, 3), (4, 6), (5, 7),
          (1, 2), (5, 6),
          (0, 4), (1, 5), (2, 6), (3, 7),
          (1, 4), (3, 6),
          (2, 4), (3, 5),
          (3, 4)]
_BMERGE8 = [(0, 4), (1, 5), (2, 6), (3, 7),
            (0, 2), (1, 3), (4, 6), (5, 7),
            (0, 1), (2, 3), (4, 5), (6, 7)]


def _sort8(v):
    v = list(v)
    for a, b in _SORT8:
        v[a], v[b] = jnp.maximum(v[a], v[b]), jnp.minimum(v[a], v[b])
    return v


def _merge_top8(A, B):
    c = [jnp.maximum(A[i], B[7 - i]) for i in range(8)]
    for a, b in _BMERGE8:
        c[a], c[b] = jnp.maximum(c[a], c[b]), jnp.minimum(c[a], c[b])
    return c


def _matmul_pack_body(x_ref, w_ref, b_ref, key_ref):
    lt = jax.lax.dot_general(
        w_ref[...], x_ref[...], (((1,), (1,)), ((), ())),
        preferred_element_type=jnp.float32,
    )
    lt = lt + b_ref[...]
    s = jax.lax.bitcast_convert_type(lt, jnp.int32)
    u = jax.lax.bitcast_convert_type(
        s ^ ((s >> 31) | jnp.int32(-(2 ** 31))), jnp.uint32
    )
    eid = jax.lax.broadcasted_iota(jnp.uint32, lt.shape, 0)
    key_ref[...] = (u & jnp.uint32(0xFFFFFFC0)) | (jnp.uint32(63) - eid)


def _packed_keys_t(x, W, b, tile, row0, rows):
    n, d = x.shape
    e = W.shape[0]
    off = row0 // tile
    return pl.pallas_call(
        _matmul_pack_body,
        grid=(rows // tile,),
        in_specs=[
            pl.BlockSpec((tile, d), lambda i: (off + i, 0)),
            pl.BlockSpec((e, d), lambda i: (0, 0)),
            pl.BlockSpec((e, 1), lambda i: (0, 0)),
        ],
        out_specs=pl.BlockSpec((e, tile), lambda i: (0, i)),
        out_shape=jax.ShapeDtypeStruct((e, rows), jnp.uint32),
    )(x, W, b.reshape(e, 1))


def _make_sc_topk(n, e):
    info = plsc.get_sparse_core_info()
    nc, ns, nl = info.num_cores, info.num_subcores, info.num_lanes
    nw = nc * ns
    assert n % (nw * nl) == 0 and e == 64
    tok_w = n // nw
    ngroups = tok_w // nl
    mesh = plsc.VectorSubcoreMesh(core_axis_name="c", subcore_axis_name="s")

    @functools.partial(
        pl.kernel, mesh=mesh,
        out_type=[
            jax.ShapeDtypeStruct((_TOPK, n), jnp.float32),
            jax.ShapeDtypeStruct((_TOPK, n), jnp.int32),
        ],
        scratch_types=[
            pltpu.VMEM((e, tok_w), jnp.uint32),
            pltpu.VMEM((_TOPK, tok_w), jnp.float32),
            pltpu.VMEM((_TOPK, tok_w), jnp.int32),
        ],
    )
    def sc_topk(key_hbm, gt_hbm, it_hbm, key_v, g_v, i_v):
        wid = lax.axis_index("s") * nc + lax.axis_index("c")
        base = wid * tok_w
        pltpu.sync_copy(key_hbm.at[:, pl.ds(base, tok_w)], key_v)

        def group_body(g, carry):
            off = g * nl

            top = _sort8([key_v[t, pl.ds(off, nl)] for t in range(8)])
            for j in range(1, 8):
                top = _merge_top8(
                    top, _sort8([key_v[8 * j + t, pl.ds(off, nl)]
                                 for t in range(8)])
                )

            vals, idxs = [], []
            for k in top:
                idxs.append((jnp.uint32(63) - (k & jnp.uint32(63)))
                            .astype(jnp.int32))
                vu = (k & jnp.uint32(0xFFFFFFC0)) | jnp.uint32(32)
                pos = vu >= jnp.uint32(0x80000000)
                sb = jnp.where(pos, vu ^ jnp.uint32(0x80000000), ~vu)
                vals.append(jax.lax.bitcast_convert_type(sb, jnp.float32))

            m = vals[0]
            exps = [jnp.exp(v - m) for v in vals]
            denom = exps[0]
            for s in exps[1:]:
                denom = denom + s
            inv = 1.0 / denom
            for k in range(_TOPK):
                g_v[k, pl.ds(off, nl)] = exps[k] * inv
                i_v[k, pl.ds(off, nl)] = idxs[k]
            return carry

        lax.fori_loop(0, ngroups, group_body, 0)
        pltpu.sync_copy(g_v, gt_hbm.at[:, pl.ds(base, tok_w)])
        pltpu.sync_copy(i_v, it_hbm.at[:, pl.ds(base, tok_w)])

    return sc_topk


def kernel(x, W, b):
    n, d = x.shape
    e = W.shape[0]
    tile = 512 if n % 512 == 0 else n
    nchunks = 1
    cs = n // nchunks
    sc = _make_sc_topk(cs, e)
    outs = []
    for c in range(nchunks):
        keys = _packed_keys_t(x, W, b, tile, c * cs, cs)
        outs.append(sc(keys))
    gt = jnp.concatenate([g for g, _ in outs], axis=1)
    it = jnp.concatenate([i for _, i in outs], axis=1)
    return gt.T, it.T.astype(jnp.int64)


# speedup vs baseline: 2.7557x; 1.0188x over previous
import functools

import jax
import jax.numpy as jnp
from jax import lax
from jax.experimental import pallas as pl
from jax.experimental.pallas import tpu as pltpu
from jax.experimental.pallas import tpu_sc as plsc

_TOPK = 8

_SORT8 = [(0, 1), (2, 3), (4, 5), (6, 7),
          (0, 2), (1, 3), (4, 6), (5, 7),
          (1, 2), (5, 6),
          (0, 4), (1, 5), (2, 6), (3, 7),
          (1, 4), (3, 6),
          (2, 4), (3, 5),
          (3, 4)]
_BMERGE8 = [(0, 4), (1, 5), (2, 6), (3, 7),
            (0, 2), (1, 3), (4, 6), (5, 7),
            (0, 1), (2, 3), (4, 5), (6, 7)]


def _sort8(v):
    v = list(v)
    for a, b in _SORT8:
        v[a], v[b] = jnp.maximum(v[a], v[b]), jnp.minimum(v[a], v[b])
    return v


def _merge_top8(A, B):
    c = [jnp.maximum(A[i], B[7 - i]) for i in range(8)]
    for a, b in _BMERGE8:
        c[a], c[b] = jnp.maximum(c[a], c[b]), jnp.minimum(c[a], c[b])
    return c


def _matmul_pack_body(x_ref, w_ref, b_ref, key_ref):
    lt = jax.lax.dot_general(
        w_ref[...], x_ref[...], (((1,), (1,)), ((), ())),
        preferred_element_type=jnp.float32,
    )
    lt = lt + b_ref[...]
    s = jax.lax.bitcast_convert_type(lt, jnp.int32)
    u = jax.lax.bitcast_convert_type(
        s ^ ((s >> 31) | jnp.int32(-(2 ** 31))), jnp.uint32
    )
    eid = jax.lax.broadcasted_iota(jnp.uint32, lt.shape, 0)
    key_ref[...] = (u & jnp.uint32(0xFFFFFFC0)) | (jnp.uint32(63) - eid)


def _packed_keys_t(x, W, b, tile, row0, rows):
    n, d = x.shape
    e = W.shape[0]
    off = row0 // tile
    return pl.pallas_call(
        _matmul_pack_body,
        grid=(rows // tile,),
        in_specs=[
            pl.BlockSpec((tile, d), lambda i: (off + i, 0)),
            pl.BlockSpec((e, d), lambda i: (0, 0)),
            pl.BlockSpec((e, 1), lambda i: (0, 0)),
        ],
        out_specs=pl.BlockSpec((e, tile), lambda i: (0, i)),
        out_shape=jax.ShapeDtypeStruct((e, rows), jnp.uint32),
    )(x, W, b.reshape(e, 1))


def _make_sc_topk(n, e):
    info = plsc.get_sparse_core_info()
    nc, ns, nl = info.num_cores, info.num_subcores, info.num_lanes
    nw = nc * ns
    assert n % (nw * nl) == 0 and e == 64
    tok_w = n // nw
    ngroups = tok_w // nl
    mesh = plsc.VectorSubcoreMesh(core_axis_name="c", subcore_axis_name="s")

    @functools.partial(
        pl.kernel, mesh=mesh,
        out_type=[
            jax.ShapeDtypeStruct((_TOPK, n), jnp.float32),
            jax.ShapeDtypeStruct((_TOPK, n), jnp.int32),
        ],
        scratch_types=[
            pltpu.VMEM((e, tok_w), jnp.uint32),
            pltpu.VMEM((_TOPK, tok_w), jnp.float32),
            pltpu.VMEM((_TOPK, tok_w), jnp.int32),
        ],
    )
    def sc_topk(key_hbm, gt_hbm, it_hbm, key_v, g_v, i_v):
        wid = lax.axis_index("s") * nc + lax.axis_index("c")
        base = wid * tok_w
        pltpu.sync_copy(key_hbm.at[:, pl.ds(base, tok_w)], key_v)

        def group_body(g, carry):
            off = g * nl

            top = _sort8([key_v[t, pl.ds(off, nl)] for t in range(8)])
            for j in range(1, 8):
                top = _merge_top8(
                    top, _sort8([key_v[8 * j + t, pl.ds(off, nl)]
                                 for t in range(8)])
                )

            vals, idxs = [], []
            for k in top:
                idxs.append((jnp.uint32(63) - (k & jnp.uint32(63)))
                            .astype(jnp.int32))
                vu = (k & jnp.uint32(0xFFFFFFC0)) | jnp.uint32(32)
                pos = vu >= jnp.uint32(0x80000000)
                sb = jnp.where(pos, vu ^ jnp.uint32(0x80000000), ~vu)
                vals.append(jax.lax.bitcast_convert_type(sb, jnp.float32))

            m = vals[0]
            exps = [jnp.exp(v - m) for v in vals]
            denom = exps[0]
            for s in exps[1:]:
                denom = denom + s
            inv = 1.0 / denom
            for k in range(_TOPK):
                g_v[k, pl.ds(off, nl)] = exps[k] * inv
                i_v[k, pl.ds(off, nl)] = idxs[k]
            return carry

        lax.fori_loop(0, ngroups, group_body, 0)
        pltpu.sync_copy(g_v, gt_hbm.at[:, pl.ds(base, tok_w)])
        pltpu.sync_copy(i_v, it_hbm.at[:, pl.ds(base, tok_w)])

    return sc_topk


def kernel(x, W, b):
    n, d = x.shape
    e = W.shape[0]
    tile = 1024 if n % 1024 == 0 else n
    nchunks = 1
    cs = n // nchunks
    sc = _make_sc_topk(cs, e)
    outs = []
    for c in range(nchunks):
        keys = _packed_keys_t(x, W, b, tile, c * cs, cs)
        outs.append(sc(keys))
    gt = jnp.concatenate([g for g, _ in outs], axis=1)
    it = jnp.concatenate([i for _, i in outs], axis=1)
    return gt.T, it.T.astype(jnp.int64)


# speedup vs baseline: 2.7579x; 1.0008x over previous
import functools

import jax
import jax.numpy as jnp
from jax import lax
from jax.experimental import pallas as pl
from jax.experimental.pallas import tpu as pltpu
from jax.experimental.pallas import tpu_sc as plsc

_TOPK = 8

_SORT8 = [(0, 1), (2, 3), (4, 5), (6, 7),
          (0, 2), (1, 3), (4, 6), (5, 7),
          (1, 2), (5, 6),
          (0, 4), (1, 5), (2, 6), (3, 7),
          (1, 4), (3, 6),
          (2, 4), (3, 5),
          (3, 4)]
_BMERGE8 = [(0, 4), (1, 5), (2, 6), (3, 7),
            (0, 2), (1, 3), (4, 6), (5, 7),
            (0, 1), (2, 3), (4, 5), (6, 7)]


def _sort8(v):
    v = list(v)
    for a, b in _SORT8:
        v[a], v[b] = jnp.maximum(v[a], v[b]), jnp.minimum(v[a], v[b])
    return v


def _merge_top8(A, B):
    c = [jnp.maximum(A[i], B[7 - i]) for i in range(8)]
    for a, b in _BMERGE8:
        c[a], c[b] = jnp.maximum(c[a], c[b]), jnp.minimum(c[a], c[b])
    return c


def _matmul_pack_body(x_ref, w_ref, b_ref, key_ref):
    lt = jax.lax.dot_general(
        w_ref[...], x_ref[...], (((1,), (1,)), ((), ())),
        preferred_element_type=jnp.float32,
    )
    lt = lt + b_ref[...]
    s = jax.lax.bitcast_convert_type(lt, jnp.int32)
    u = jax.lax.bitcast_convert_type(
        s ^ ((s >> 31) | jnp.int32(-(2 ** 31))), jnp.uint32
    )
    eid = jax.lax.broadcasted_iota(jnp.uint32, lt.shape, 0)
    key_ref[...] = (u & jnp.uint32(0xFFFFFFC0)) | (jnp.uint32(63) - eid)


def _packed_keys_t(x, W, b, tile, row0, rows):
    n, d = x.shape
    e = W.shape[0]
    off = row0 // tile
    return pl.pallas_call(
        _matmul_pack_body,
        grid=(rows // tile,),
        in_specs=[
            pl.BlockSpec((tile, d), lambda i: (off + i, 0)),
            pl.BlockSpec((e, d), lambda i: (0, 0)),
            pl.BlockSpec((e, 1), lambda i: (0, 0)),
        ],
        out_specs=pl.BlockSpec((e, tile), lambda i: (0, i)),
        out_shape=jax.ShapeDtypeStruct((e, rows), jnp.uint32),
        compiler_params=pltpu.CompilerParams(
            dimension_semantics=("parallel",)),
    )(x, W, b.reshape(e, 1))


def _make_sc_topk(n, e):
    info = plsc.get_sparse_core_info()
    nc, ns, nl = info.num_cores, info.num_subcores, info.num_lanes
    nw = nc * ns
    assert n % (nw * nl) == 0 and e == 64
    tok_w = n // nw
    ngroups = tok_w // nl
    mesh = plsc.VectorSubcoreMesh(core_axis_name="c", subcore_axis_name="s")

    @functools.partial(
        pl.kernel, mesh=mesh,
        out_type=[
            jax.ShapeDtypeStruct((_TOPK, n), jnp.float32),
            jax.ShapeDtypeStruct((_TOPK, n), jnp.int32),
        ],
        scratch_types=[
            pltpu.VMEM((e, tok_w), jnp.uint32),
            pltpu.VMEM((_TOPK, tok_w), jnp.float32),
            pltpu.VMEM((_TOPK, tok_w), jnp.int32),
        ],
    )
    def sc_topk(key_hbm, gt_hbm, it_hbm, key_v, g_v, i_v):
        wid = lax.axis_index("s") * nc + lax.axis_index("c")
        base = wid * tok_w
        pltpu.sync_copy(key_hbm.at[:, pl.ds(base, tok_w)], key_v)

        def group_body(g, carry):
            off = g * nl

            top = _sort8([key_v[t, pl.ds(off, nl)] for t in range(8)])
            for j in range(1, 8):
                top = _merge_top8(
                    top, _sort8([key_v[8 * j + t, pl.ds(off, nl)]
                                 for t in range(8)])
                )

            vals, idxs = [], []
            for k in top:
                idxs.append((jnp.uint32(63) - (k & jnp.uint32(63)))
                            .astype(jnp.int32))
                vu = (k & jnp.uint32(0xFFFFFFC0)) | jnp.uint32(32)
                pos = vu >= jnp.uint32(0x80000000)
                sb = jnp.where(pos, vu ^ jnp.uint32(0x80000000), ~vu)
                vals.append(jax.lax.bitcast_convert_type(sb, jnp.float32))

            m = vals[0]
            exps = [jnp.exp(v - m) for v in vals]
            denom = exps[0]
            for s in exps[1:]:
                denom = denom + s
            inv = 1.0 / denom
            for k in range(_TOPK):
                g_v[k, pl.ds(off, nl)] = exps[k] * inv
                i_v[k, pl.ds(off, nl)] = idxs[k]
            return carry

        lax.fori_loop(0, ngroups, group_body, 0)
        pltpu.sync_copy(g_v, gt_hbm.at[:, pl.ds(base, tok_w)])
        pltpu.sync_copy(i_v, it_hbm.at[:, pl.ds(base, tok_w)])

    return sc_topk


def kernel(x, W, b):
    n, d = x.shape
    e = W.shape[0]
    tile = 1024 if n % 1024 == 0 else n
    nchunks = 1
    cs = n // nchunks
    sc = _make_sc_topk(cs, e)
    outs = []
    for c in range(nchunks):
        keys = _packed_keys_t(x, W, b, tile, c * cs, cs)
        outs.append(sc(keys))
    gt = jnp.concatenate([g for g, _ in outs], axis=1)
    it = jnp.concatenate([i for _, i in outs], axis=1)
    return gt.T, it.T.astype(jnp.int64)


# speedup vs baseline: 2.7598x; 1.0007x over previous
import functools

import jax
import jax.numpy as jnp
from jax import lax
from jax.experimental import pallas as pl
from jax.experimental.pallas import tpu as pltpu
from jax.experimental.pallas import tpu_sc as plsc

_TOPK = 8

_SORT8 = [(0, 1), (2, 3), (4, 5), (6, 7),
          (0, 2), (1, 3), (4, 6), (5, 7),
          (1, 2), (5, 6),
          (0, 4), (1, 5), (2, 6), (3, 7),
          (1, 4), (3, 6),
          (2, 4), (3, 5),
          (3, 4)]
_BMERGE8 = [(0, 4), (1, 5), (2, 6), (3, 7),
            (0, 2), (1, 3), (4, 6), (5, 7),
            (0, 1), (2, 3), (4, 5), (6, 7)]


def _sort8(v):
    v = list(v)
    for a, b in _SORT8:
        v[a], v[b] = jnp.maximum(v[a], v[b]), jnp.minimum(v[a], v[b])
    return v


def _merge_top8(A, B):
    c = [jnp.maximum(A[i], B[7 - i]) for i in range(8)]
    for a, b in _BMERGE8:
        c[a], c[b] = jnp.maximum(c[a], c[b]), jnp.minimum(c[a], c[b])
    return c


def _matmul_pack_body(x_ref, w_ref, b_ref, key_ref):
    lt = jax.lax.dot_general(
        w_ref[...], x_ref[...], (((1,), (1,)), ((), ())),
        preferred_element_type=jnp.float32,
    )
    lt = lt + b_ref[...]
    s = jax.lax.bitcast_convert_type(lt, jnp.int32)
    m = s ^ ((s >> 31) & jnp.int32(0x7FFFFFFF))
    eid = jax.lax.broadcasted_iota(jnp.int32, lt.shape, 0)
    kk = (m & jnp.int32(-64)) | (jnp.int32(63) - eid)
    blocks = _sort8([kk[8 * t:8 * t + 8] for t in range(8)])
    for t in range(8):
        key_ref[8 * t:8 * t + 8, :] = blocks[t]


def _packed_keys_t(x, W, b, tile, row0, rows):
    n, d = x.shape
    e = W.shape[0]
    off = row0 // tile
    return pl.pallas_call(
        _matmul_pack_body,
        grid=(rows // tile,),
        in_specs=[
            pl.BlockSpec((tile, d), lambda i: (off + i, 0)),
            pl.BlockSpec((e, d), lambda i: (0, 0)),
            pl.BlockSpec((e, 1), lambda i: (0, 0)),
        ],
        out_specs=pl.BlockSpec((e, tile), lambda i: (0, i)),
        out_shape=jax.ShapeDtypeStruct((e, rows), jnp.int32),
    )(x, W, b.reshape(e, 1))


def _make_sc_topk(n, e):
    info = plsc.get_sparse_core_info()
    nc, ns, nl = info.num_cores, info.num_subcores, info.num_lanes
    nw = nc * ns
    assert n % (nw * nl) == 0 and e == 64
    tok_w = n // nw
    ngroups = tok_w // nl
    mesh = plsc.VectorSubcoreMesh(core_axis_name="c", subcore_axis_name="s")

    @functools.partial(
        pl.kernel, mesh=mesh,
        out_type=[
            jax.ShapeDtypeStruct((_TOPK, n), jnp.float32),
            jax.ShapeDtypeStruct((_TOPK, n), jnp.int32),
        ],
        scratch_types=[
            pltpu.VMEM((e, tok_w), jnp.int32),
            pltpu.VMEM((_TOPK, tok_w), jnp.float32),
            pltpu.VMEM((_TOPK, tok_w), jnp.int32),
        ],
    )
    def sc_topk(key_hbm, gt_hbm, it_hbm, key_v, g_v, i_v):
        wid = lax.axis_index("s") * nc + lax.axis_index("c")
        base = wid * tok_w
        pltpu.sync_copy(key_hbm.at[:, pl.ds(base, tok_w)], key_v)

        def group_body(g, carry):
            off = g * nl

            top = [key_v[8 * t, pl.ds(off, nl)] for t in range(8)]
            for j in range(1, 8):
                top = _merge_top8(
                    top, [key_v[8 * t + j, pl.ds(off, nl)]
                          for t in range(8)]
                )

            vals, idxs = [], []
            for k in top:
                idxs.append(jnp.int32(63) - (k & jnp.int32(63)))
                vu = (k & jnp.int32(-64)) | jnp.int32(32)
                sb = vu ^ ((vu >> 31) & jnp.int32(0x7FFFFFFF))
                vals.append(jax.lax.bitcast_convert_type(sb, jnp.float32))

            m = vals[0]
            exps = [jnp.exp(v - m) for v in vals]
            denom = exps[0]
            for s in exps[1:]:
                denom = denom + s
            inv = 1.0 / denom
            for k in range(_TOPK):
                g_v[k, pl.ds(off, nl)] = exps[k] * inv
                i_v[k, pl.ds(off, nl)] = idxs[k]
            return carry

        lax.fori_loop(0, ngroups, group_body, 0)
        pltpu.sync_copy(g_v, gt_hbm.at[:, pl.ds(base, tok_w)])
        pltpu.sync_copy(i_v, it_hbm.at[:, pl.ds(base, tok_w)])

    return sc_topk


def kernel(x, W, b):
    n, d = x.shape
    e = W.shape[0]
    tile = 1024 if n % 1024 == 0 else n
    nchunks = 1
    cs = n // nchunks
    sc = _make_sc_topk(cs, e)
    outs = []
    for c in range(nchunks):
        keys = _packed_keys_t(x, W, b, tile, c * cs, cs)
        outs.append(sc(keys))
    gt = jnp.concatenate([g for g, _ in outs], axis=1)
    it = jnp.concatenate([i for _, i in outs], axis=1)
    return gt.T, it.T.astype(jnp.int64)
